# baseline (speedup 1.0000x reference)
import functools

import jax
import jax.numpy as jnp
from jax.experimental import pallas as pl
from jax.experimental.pallas import tpu as pltpu

HIDDEN = 5120
NUM_EXPERTS = 128
TOP_K = 4
MIN_TEMP = 0.1
EPS = 1e-08

BLK = 512
HALF = BLK // 2


def _top4(s, iota):
    idxs = []
    vals = []
    for _ in range(TOP_K):
        mx = jnp.max(s, axis=-1, keepdims=True)
        pick = jnp.min(
            jnp.where(s == mx, iota, NUM_EXPERTS), axis=-1, keepdims=True
        )
        vals.append(mx)
        idxs.append(pick)
        s = jnp.where(iota == pick, -1.0, s)
    total = vals[0] + vals[1] + vals[2] + vals[3] + EPS
    return jnp.concatenate(idxs, axis=1), jnp.concatenate(vals, axis=1) / total


def _gate_kernel(t_ref, xa_ref, xb_ref, w_ref, idx_ref, wt_ref):
    inv_t = 1.0 / t_ref[0]
    wb = w_ref[...]
    iota = jax.lax.broadcasted_iota(jnp.int32, (HALF, NUM_EXPERTS), 1)
    for h, x_ref in enumerate((xa_ref, xb_ref)):
        xh = x_ref[...].astype(jnp.bfloat16)
        logits = jax.lax.dot_general(
            xh, wb,
            dimension_numbers=(((1,), (0,)), ((), ())),
            preferred_element_type=jnp.float32,
        )
        ls = logits * inv_t
        m = jnp.max(ls, axis=-1, keepdims=True)
        e = jnp.exp(ls - m)
        denom = jnp.sum(e, axis=-1, keepdims=True)
        s = e / denom
        idx, wt = _top4(s, iota)
        sl = pl.ds(h * HALF, HALF)
        idx_ref[sl, :] = idx
        wt_ref[sl, :] = wt


@functools.partial(jax.jit, static_argnames=())
def kernel(x, gate_w, temperature):
    n_rows = x.shape[0]
    t = jnp.maximum(jax.nn.softplus(temperature), MIN_TEMP).reshape((1,))
    wt = gate_w.T.astype(jnp.bfloat16)
    grid = (n_rows // BLK,)
    idx, w = pl.pallas_call(
        _gate_kernel,
        grid=grid,
        in_specs=[
            pl.BlockSpec(memory_space=pltpu.SMEM),
            pl.BlockSpec((HALF, HIDDEN), lambda i: (2 * i, 0)),
            pl.BlockSpec((HALF, HIDDEN), lambda i: (2 * i + 1, 0)),
            pl.BlockSpec((HIDDEN, NUM_EXPERTS), lambda i: (0, 0)),
        ],
        out_specs=[
            pl.BlockSpec((BLK, TOP_K), lambda i: (i, 0)),
            pl.BlockSpec((BLK, TOP_K), lambda i: (i, 0)),
        ],
        out_shape=[
            jax.ShapeDtypeStruct((n_rows, TOP_K), jnp.int32),
            jax.ShapeDtypeStruct((n_rows, TOP_K), jnp.float32),
        ],
    )(t, x, x, wt)
    return idx, w


# speedup vs baseline: 1.0413x; 1.0413x over previous
import functools

import jax
import jax.numpy as jnp
from jax.experimental import pallas as pl
from jax.experimental.pallas import tpu as pltpu

HIDDEN = 5120
NUM_EXPERTS = 128
TOP_K = 4
MIN_TEMP = 0.1
EPS = 1e-08

BLK = 1024
HALF = BLK // 2


def _top4(s, iota):
    idxs = []
    vals = []
    for _ in range(TOP_K):
        mx = jnp.max(s, axis=-1, keepdims=True)
        pick = jnp.min(
            jnp.where(s == mx, iota, NUM_EXPERTS), axis=-1, keepdims=True
        )
        vals.append(mx)
        idxs.append(pick)
        s = jnp.where(iota == pick, -1.0, s)
    total = vals[0] + vals[1] + vals[2] + vals[3] + EPS
    return jnp.concatenate(idxs, axis=1), jnp.concatenate(vals, axis=1) / total


def _gate_kernel(t_ref, xa_ref, xb_ref, w_ref, idx_ref, wt_ref):
    inv_t = 1.0 / t_ref[0]
    wb = w_ref[...]
    iota = jax.lax.broadcasted_iota(jnp.int32, (HALF, NUM_EXPERTS), 1)
    for h, x_ref in enumerate((xa_ref, xb_ref)):
        xh = x_ref[...].astype(jnp.bfloat16)
        logits = jax.lax.dot_general(
            xh, wb,
            dimension_numbers=(((1,), (0,)), ((), ())),
            preferred_element_type=jnp.float32,
        )
        ls = logits * inv_t
        m = jnp.max(ls, axis=-1, keepdims=True)
        e = jnp.exp(ls - m)
        denom = jnp.sum(e, axis=-1, keepdims=True)
        s = e / denom
        idx, wt = _top4(s, iota)
        sl = pl.ds(h * HALF, HALF)
        idx_ref[sl, :] = idx
        wt_ref[sl, :] = wt


@functools.partial(jax.jit, static_argnames=())
def kernel(x, gate_w, temperature):
    n_rows = x.shape[0]
    t = jnp.maximum(jax.nn.softplus(temperature), MIN_TEMP).reshape((1,))
    wt = gate_w.T.astype(jnp.bfloat16)
    grid = (n_rows // BLK,)
    idx, w = pl.pallas_call(
        _gate_kernel,
        grid=grid,
        in_specs=[
            pl.BlockSpec(memory_space=pltpu.SMEM),
            pl.BlockSpec((HALF, HIDDEN), lambda i: (2 * i, 0)),
            pl.BlockSpec((HALF, HIDDEN), lambda i: (2 * i + 1, 0)),
            pl.BlockSpec((HIDDEN, NUM_EXPERTS), lambda i: (0, 0)),
        ],
        out_specs=[
            pl.BlockSpec((BLK, TOP_K), lambda i: (i, 0)),
            pl.BlockSpec((BLK, TOP_K), lambda i: (i, 0)),
        ],
        out_shape=[
            jax.ShapeDtypeStruct((n_rows, TOP_K), jnp.int32),
            jax.ShapeDtypeStruct((n_rows, TOP_K), jnp.float32),
        ],
    )(t, x, x, wt)
    return idx, w


# speedup vs baseline: 1.0787x; 1.0359x over previous
import functools

import jax
import jax.numpy as jnp
from jax.experimental import pallas as pl
from jax.experimental.pallas import tpu as pltpu

HIDDEN = 5120
NUM_EXPERTS = 128
TOP_K = 4
MIN_TEMP = 0.1
EPS = 1e-08

BLK = 1024
HALF = BLK // 2


def _top4(s, iota):
    idxs = []
    vals = []
    for _ in range(TOP_K):
        mx = jnp.max(s, axis=-1, keepdims=True)
        pick = jnp.min(
            jnp.where(s == mx, iota, NUM_EXPERTS), axis=-1, keepdims=True
        )
        vals.append(mx)
        idxs.append(pick)
        s = jnp.where(iota == pick, -1.0, s)
    total = vals[0] + vals[1] + vals[2] + vals[3] + EPS
    return jnp.concatenate(idxs, axis=1), jnp.concatenate(vals, axis=1) / total


def _gate_kernel(t_ref, xa_ref, xb_ref, w_ref, idx_ref, wt_ref):
    inv_t = 1.0 / t_ref[0]
    wb = w_ref[...].astype(jnp.bfloat16)
    iota = jax.lax.broadcasted_iota(jnp.int32, (HALF, NUM_EXPERTS), 1)
    for h, x_ref in enumerate((xa_ref, xb_ref)):
        xh = x_ref[...].astype(jnp.bfloat16)
        logits = jax.lax.dot_general(
            xh, wb,
            dimension_numbers=(((1,), (1,)), ((), ())),
            preferred_element_type=jnp.float32,
        )
        ls = logits * inv_t
        m = jnp.max(ls, axis=-1, keepdims=True)
        e = jnp.exp(ls - m)
        denom = jnp.sum(e, axis=-1, keepdims=True)
        s = e / denom
        idx, wt = _top4(s, iota)
        sl = pl.ds(h * HALF, HALF)
        idx_ref[sl, :] = idx
        wt_ref[sl, :] = wt


@functools.partial(jax.jit, static_argnames=())
def kernel(x, gate_w, temperature):
    n_rows = x.shape[0]
    t = jnp.maximum(jax.nn.softplus(temperature), MIN_TEMP).reshape((1,))
    grid = (n_rows // BLK,)
    idx, w = pl.pallas_call(
        _gate_kernel,
        grid=grid,
        in_specs=[
            pl.BlockSpec(memory_space=pltpu.SMEM),
            pl.BlockSpec((HALF, HIDDEN), lambda i: (2 * i, 0)),
            pl.BlockSpec((HALF, HIDDEN), lambda i: (2 * i + 1, 0)),
            pl.BlockSpec((NUM_EXPERTS, HIDDEN), lambda i: (0, 0)),
        ],
        out_specs=[
            pl.BlockSpec((BLK, TOP_K), lambda i: (i, 0)),
            pl.BlockSpec((BLK, TOP_K), lambda i: (i, 0)),
        ],
        out_shape=[
            jax.ShapeDtypeStruct((n_rows, TOP_K), jnp.int32),
            jax.ShapeDtypeStruct((n_rows, TOP_K), jnp.float32),
        ],
    )(t, x, x, gate_w)
    return idx, w


# speedup vs baseline: 1.1997x; 1.1121x over previous
import functools

import jax
import jax.numpy as jnp
from jax.experimental import pallas as pl
from jax.experimental.pallas import tpu as pltpu

HIDDEN = 5120
NUM_EXPERTS = 128
TOP_K = 4
MIN_TEMP = 0.1
EPS = 1e-08

BLK = 1024
HALF = BLK // 2


def _top4(s, iota):
    idxs = []
    vals = []
    for _ in range(TOP_K):
        mx = jnp.max(s, axis=-1, keepdims=True)
        pick = jnp.min(
            jnp.where(s == mx, iota, NUM_EXPERTS), axis=-1, keepdims=True
        )
        vals.append(mx)
        idxs.append(pick)
        s = jnp.where(iota == pick, -1.0, s)
    total = vals[0] + vals[1] + vals[2] + vals[3] + EPS
    idx = jnp.concatenate(idxs, axis=1).T
    wt = (jnp.concatenate(vals, axis=1) / total).T
    return idx, wt


def _gate_kernel(t_ref, xa_ref, xb_ref, w_ref, idx_ref, wt_ref):
    inv_t = 1.0 / t_ref[0]
    wb = w_ref[...].astype(jnp.bfloat16)
    iota = jax.lax.broadcasted_iota(jnp.int32, (HALF, NUM_EXPERTS), 1)
    for h, x_ref in enumerate((xa_ref, xb_ref)):
        xh = x_ref[...].astype(jnp.bfloat16)
        logits = jax.lax.dot_general(
            xh, wb,
            dimension_numbers=(((1,), (1,)), ((), ())),
            preferred_element_type=jnp.float32,
        )
        ls = logits * inv_t
        m = jnp.max(ls, axis=-1, keepdims=True)
        e = jnp.exp(ls - m)
        denom = jnp.sum(e, axis=-1, keepdims=True)
        s = e / denom
        idx, wt = _top4(s, iota)
        sl = pl.ds(h * HALF, HALF)
        idx_ref[:, sl] = idx
        wt_ref[:, sl] = wt


@functools.partial(jax.jit, static_argnames=())
def kernel(x, gate_w, temperature):
    n_rows = x.shape[0]
    t = jnp.maximum(jax.nn.softplus(temperature), MIN_TEMP).reshape((1,))
    grid = (n_rows // BLK,)
    idx, w = pl.pallas_call(
        _gate_kernel,
        grid=grid,
        in_specs=[
            pl.BlockSpec(memory_space=pltpu.SMEM),
            pl.BlockSpec((HALF, HIDDEN), lambda i: (2 * i, 0)),
            pl.BlockSpec((HALF, HIDDEN), lambda i: (2 * i + 1, 0)),
            pl.BlockSpec((NUM_EXPERTS, HIDDEN), lambda i: (0, 0)),
        ],
        out_specs=[
            pl.BlockSpec((TOP_K, BLK), lambda i: (0, i)),
            pl.BlockSpec((TOP_K, BLK), lambda i: (0, i)),
        ],
        out_shape=[
            jax.ShapeDtypeStruct((TOP_K, n_rows), jnp.int32),
            jax.ShapeDtypeStruct((TOP_K, n_rows), jnp.float32),
        ],
    )(t, x, x, gate_w)
    return idx.T, w.T
